# baseline (speedup 1.0000x reference)
import functools

import jax
import jax.numpy as jnp
from jax import lax
from jax.experimental import pallas as pl
from jax.experimental.pallas import tpu as pltpu
from jax.experimental.pallas import tpu_sc as plsc

N = 100000
E = 1600000
IN_DIM = 4
HID = 64
HEADS = 2
EMBED = 64
G = 64
F = HEADS * HID

R = 512
NB = 196
NBP = 272
NPAD = NB * R
BN = 2048
E2 = 1703936
ECH = 1024
NCH = E2 // ECH
W = 32
CPT = NCH // W
K = 128
EPAD = E2 + NBP * K + ECH
BPT = 7
AW = 144

_mesh = plsc.VectorSubcoreMesh(core_axis_name="c", subcore_axis_name="s")


def _z16i():
    return jnp.zeros((16,), jnp.int32)


def _z16f():
    return jnp.zeros((16,), jnp.float32)


def _wid():
    return lax.axis_index("s") * 2 + lax.axis_index("c")


def _lanes():
    return lax.iota(jnp.int32, 16)


@functools.partial(
    pl.kernel, mesh=_mesh,
    out_type=jax.ShapeDtypeStruct((W * NBP,), jnp.int32),
    scratch_types=[pltpu.VMEM((ECH,), jnp.int32), pltpu.VMEM((NBP,), jnp.int32)],
)
def _phase_hist(dst_hbm, counts_hbm, dbuf, cntb):
    wid = _wid()
    lanes = _lanes()
    one0 = jnp.where(lanes == 0, 1, 0)
    for j in range(NBP // 16):
        cntb[pl.ds(j * 16, 16)] = _z16i()

    def chunk(c, _):
        pltpu.sync_copy(dst_hbm.at[pl.ds((wid * CPT + c) * ECH, ECH)], dbuf)

        def grp(g, _):
            bv = dbuf[pl.ds(g * 16, 16)] >> 9
            for l in range(16):
                plsc.addupdate(cntb.at[pl.ds(bv[l], 16)], one0)
            return 0
        return lax.fori_loop(0, ECH // 16, grp, 0)
    lax.fori_loop(0, CPT, chunk, 0)
    pltpu.sync_copy(cntb, counts_hbm.at[pl.ds(pl.multiple_of(wid * NBP, 16), NBP)])


@functools.partial(
    pl.kernel, mesh=_mesh,
    out_type=(
        jax.ShapeDtypeStruct((W * NBP,), jnp.int32),
        jax.ShapeDtypeStruct((NBP,), jnp.int32),
        jax.ShapeDtypeStruct((NBP,), jnp.int32),
    ),
    scratch_types=[
        pltpu.VMEM((W * NBP,), jnp.int32),
        pltpu.VMEM((W * NBP,), jnp.int32),
        pltpu.VMEM((NBP,), jnp.int32),
        pltpu.VMEM((NBP,), jnp.int32),
        pltpu.VMEM((NBP,), jnp.int32),
    ],
)
def _phase_prefix(counts_hbm, cur_hbm, off_hbm, tot_hbm, cnts, curb, offb, totb,
                  runb):
    @pl.when(_wid() == 0)
    def _():
        pltpu.sync_copy(counts_hbm, cnts)
        nj = NBP // 16
        for j in range(nj):
            runb[pl.ds(j * 16, 16)] = _z16i()

        def tile(t, _):
            tb = t * NBP
            for j in range(nj):
                r = runb[pl.ds(j * 16, 16)]
                curb[pl.ds(tb + j * 16, 16)] = r
                runb[pl.ds(j * 16, 16)] = r + cnts[pl.ds(tb + j * 16, 16)]
            return 0
        lax.fori_loop(0, W, tile, 0)

        lanes = _lanes()
        carry = jnp.int32(0)
        for j in range(nj):
            tot = runb[pl.ds(j * 16, 16)]
            padded = (tot + (K - 1)) & ~(K - 1)
            excl = _z16i()
            for l in range(16):
                excl = jnp.where(lanes == l, carry, excl)
                carry = carry + padded[l]
            offb[pl.ds(j * 16, 16)] = excl
            totb[pl.ds(j * 16, 16)] = tot

        def tile2(t, _):
            tb = t * NBP
            for j in range(nj):
                curb[pl.ds(tb + j * 16, 16)] = (
                    curb[pl.ds(tb + j * 16, 16)] + offb[pl.ds(j * 16, 16)])
            return 0
        lax.fori_loop(0, W, tile2, 0)
        pltpu.sync_copy(curb, cur_hbm)
        pltpu.sync_copy(offb, off_hbm)
        pltpu.sync_copy(totb, tot_hbm)


@functools.partial(
    pl.kernel, mesh=_mesh,
    out_type=jax.ShapeDtypeStruct((EPAD,), jnp.int32),
    scratch_types=[
        pltpu.VMEM((ECH,), jnp.int32),
        pltpu.VMEM((ECH,), jnp.int32),
        pltpu.VMEM((NBP,), jnp.int32),
        pltpu.VMEM((K,), jnp.int32),
        pltpu.VMEM((K,), jnp.int32),
    ],
)
def _phase_scatter(src_hbm, dst_hbm, cur_hbm, ebuf_hbm, sbuf, dbuf, curb, idxw,
                   valw):
    wid = _wid()
    lanes = _lanes()
    one0 = jnp.where(lanes == 0, 1, 0)
    pltpu.sync_copy(cur_hbm.at[pl.ds(pl.multiple_of(wid * NBP, 16), NBP)], curb)

    def chunk(c, _):
        base = (wid * CPT + c) * ECH
        pltpu.sync_copy(src_hbm.at[pl.ds(base, ECH)], sbuf)
        pltpu.sync_copy(dst_hbm.at[pl.ds(base, ECH)], dbuf)

        def group(g, _):
            def sub(h, _):
                ii = g * K + h * 16
                dv = dbuf[pl.ds(ii, 16)]
                sv = sbuf[pl.ds(ii, 16)]
                bv = dv >> 9
                vv = (sv << 9) | (dv & (R - 1))
                db = _z16i()
                for m in range(15):
                    db = db + jnp.where((bv == bv[m]) & (lanes > m), 1, 0)
                cbase = _z16i()
                for l in range(16):
                    cbase = jnp.where(lanes == l,
                                      curb[pl.ds(bv[l], 16)][0], cbase)
                for l in range(16):
                    plsc.addupdate(curb.at[pl.ds(bv[l], 16)], one0)
                idxw[pl.ds(h * 16, 16)] = cbase + db
                valw[pl.ds(h * 16, 16)] = vv
                return 0
            lax.fori_loop(0, K // 16, sub, 0)
            pltpu.sync_copy(valw, ebuf_hbm.at[idxw])
            return 0
        return lax.fori_loop(0, ECH // K, group, 0)
    lax.fori_loop(0, CPT, chunk, 0)


@functools.partial(
    pl.kernel, mesh=_mesh,
    out_type=jax.ShapeDtypeStruct((NPAD, F), jnp.float32),
    scratch_types=[
        pltpu.VMEM((R * AW,), jnp.float32),
        pltpu.VMEM((K, F), jnp.float32),
        pltpu.VMEM((K, F), jnp.float32),
        pltpu.VMEM((ECH,), jnp.int32),
        pltpu.VMEM((K,), jnp.int32),
        pltpu.VMEM((K,), jnp.int32),
        pltpu.VMEM((K,), jnp.int32),
        pltpu.VMEM((K,), jnp.int32),
        pltpu.VMEM((K + 16,), jnp.int32),
        pltpu.VMEM((K + 16,), jnp.int32),
        pltpu.VMEM((K,), jnp.float32),
        pltpu.VMEM((K,), jnp.float32),
        pltpu.VMEM((K,), jnp.float32),
        pltpu.VMEM((K,), jnp.float32),
        pltpu.VMEM((K,), jnp.float32),
        pltpu.VMEM((K,), jnp.float32),
        pltpu.VMEM((K,), jnp.float32),
        pltpu.VMEM((K,), jnp.float32),
        pltpu.VMEM((K + 16,), jnp.float32),
        pltpu.VMEM((K + 16,), jnp.float32),
        pltpu.VMEM((NBP,), jnp.int32),
        pltpu.VMEM((NBP,), jnp.int32),
        pltpu.SemaphoreType.DMA,
        pltpu.SemaphoreType.DMA,
        pltpu.SemaphoreType.DMA,
        pltpu.SemaphoreType.DMA,
    ],
)
def _edge_pass(h_hbm, as0_hbm, as1_hbm, ad0_hbm, ad1_hbm, ebuf_hbm, off_hbm,
               tot_hbm, out_hbm,
               acc, rows0, rows1, pks, ix0, ix1, dg0, dg1, dl0, dl1,
               sa00, sa01, sa10, sa11, da00, da01, da10, da11,
               ee0b, ee1b, offv, totv, semr0, semr1, sems0, sems1):
    rowsL = (rows0, rows1)
    ixL = (ix0, ix1)
    dgL = (dg0, dg1)
    dlL = (dl0, dl1)
    as0L = (sa00, sa01)
    as1L = (sa10, sa11)
    ad0L = (da00, da01)
    ad1L = (da10, da11)
    semrL = (semr0, semr1)
    semsL = (sems0, sems1)
    wid = _wid()
    lanes = _lanes()
    pltpu.sync_copy(off_hbm, offv)
    pltpu.sync_copy(tot_hbm, totv)

    def bucket_iter(bi, _):
        b = wid + W * bi

        @pl.when(b < NB)
        def _():
            base = pl.multiple_of(b * R, R)

            def zero(k, _):
                acc[pl.ds(k * 16, 16)] = _z16f()
                return 0
            lax.fori_loop(0, R * AW // 16, zero, 0, unroll=8)

            off = offv[pl.ds(b, 16)][0]
            cnt = totv[pl.ds(b, 16)][0]
            nb_batches = (cnt + K - 1) >> 7

            def fetch(jb, par):
                @pl.when((jb & 7) == 0)
                def _():
                    pltpu.sync_copy(
                        ebuf_hbm.at[pl.ds(
                            pl.multiple_of(off + jb * K, ECH), ECH)], pks)
                sb = (jb & 7) * K
                for g in range(K // 16):
                    pkv = pks[pl.ds(sb + g * 16, 16)]
                    dl16 = pkv & (R - 1)
                    ixL[par][pl.ds(g * 16, 16)] = pkv >> 9
                    dlL[par][pl.ds(g * 16, 16)] = dl16
                    dgL[par][pl.ds(g * 16, 16)] = dl16 + base
                pltpu.async_copy(h_hbm.at[ixL[par]], rowsL[par], semrL[par])
                pltpu.async_copy(as0_hbm.at[ixL[par]], as0L[par], semsL[par])
                pltpu.async_copy(as1_hbm.at[ixL[par]], as1L[par], semsL[par])
                pltpu.async_copy(ad0_hbm.at[dgL[par]], ad0L[par], semsL[par])
                pltpu.async_copy(ad1_hbm.at[dgL[par]], ad1L[par], semsL[par])

            def waitb(par):
                pltpu.make_async_copy(
                    h_hbm.at[ixL[par]], rowsL[par], semrL[par]).wait()
                pltpu.make_async_copy(
                    as0_hbm.at[ixL[par]], as0L[par], semsL[par]).wait()
                pltpu.make_async_copy(
                    as1_hbm.at[ixL[par]], as1L[par], semsL[par]).wait()
                pltpu.make_async_copy(
                    ad0_hbm.at[dgL[par]], ad0L[par], semsL[par]).wait()
                pltpu.make_async_copy(
                    ad1_hbm.at[dgL[par]], ad1L[par], semsL[par]).wait()

            def compute(jb, par):
                rows = rowsL[par]
                dlv = dlL[par]
                for g in range(K // 16):
                    lv = lanes + g * 16
                    valid = (jb * K + lv) < cnt
                    e0 = (as0L[par][pl.ds(g * 16, 16)]
                          + ad0L[par][pl.ds(g * 16, 16)])
                    e1 = (as1L[par][pl.ds(g * 16, 16)]
                          + ad1L[par][pl.ds(g * 16, 16)])
                    e0 = jnp.where(e0 > 0, e0, 0.2 * e0)
                    e1 = jnp.where(e1 > 0, e1, 0.2 * e1)
                    ee0b[pl.ds(g * 16, 16)] = jnp.where(valid, jnp.exp(e0), 0.0)
                    ee1b[pl.ds(g * 16, 16)] = jnp.where(valid, jnp.exp(e1), 0.0)

                def edge(e, _):
                    dl = dlv[pl.ds(e, 16)][0]
                    s0 = ee0b[pl.ds(e, 16)][0]
                    s1 = ee1b[pl.ds(e, 16)][0]
                    ebase = dl * AW
                    vs = [rows[e, pl.ds(j * 16, 16)] for j in range(F // 16)]
                    ms = [vs[j] * (s0 if j < 4 else s1) for j in range(F // 16)]
                    for j in range(F // 16):
                        plsc.addupdate(acc.at[pl.ds(ebase + j * 16, 16)], ms[j])
                    wv = jnp.where(lanes == 0, s0,
                                   jnp.where(lanes == 1, s1, 0.0))
                    plsc.addupdate(acc.at[pl.ds(ebase + F, 16)], wv)
                    return 0
                lax.fori_loop(0, K, edge, 0, unroll=2)

            @pl.when(nb_batches > 0)
            def _prime():
                fetch(0, 0)

            def pair(jp, _):
                for par in range(2):
                    jb = jp * 2 + par

                    @pl.when(jb < nb_batches)
                    def _(jb=jb, par=par):
                        waitb(par)

                        @pl.when(jb + 1 < nb_batches)
                        def _():
                            fetch(jb + 1, 1 - par)
                        compute(jb, par)
                return 0
            lax.fori_loop(0, (nb_batches + 1) >> 1, pair, 0)

            def flush_chunk(cc, _):
                def node(rr, _):
                    nn = cc * K + rr
                    wv = acc[pl.ds(nn * AW + F, 16)]
                    inv = 1.0 / jnp.maximum(wv, 1e-30)
                    i0 = inv[0]
                    i1 = inv[1]
                    for j in range(F // 16):
                        s = i0 if j < 4 else i1
                        rows0[rr, pl.ds(j * 16, 16)] = (
                            acc[pl.ds(nn * AW + j * 16, 16)] * s)
                    return 0
                lax.fori_loop(0, K, node, 0, unroll=2)
                pltpu.sync_copy(
                    rows0,
                    out_hbm.at[pl.ds(pl.multiple_of(base + cc * K, K), K)])
                return 0
            lax.fori_loop(0, R // K, flush_chunk, 0)
        return 0
    lax.fori_loop(0, BPT, bucket_iter, 0)


def _tc_proj_kernel(x_ref, w_ref, av_ref, b_ref, h_ref, as0_ref, as1_ref,
                    ad0_ref, ad1_ref, *, apply_elu):
    x = x_ref[...]
    if apply_elu:
        x = x + b_ref[...][None, :]
        x = jnp.where(x > 0, x, jnp.exp(jnp.minimum(x, 0.0)) - 1.0)
    h = jnp.dot(x, w_ref[...], preferred_element_type=jnp.float32)
    h_ref[...] = h
    p = jnp.dot(h, av_ref[...], preferred_element_type=jnp.float32)
    as0_ref[...] = p[:, 0]
    as1_ref[...] = p[:, 1]
    ad0_ref[...] = p[:, 2]
    ad1_ref[...] = p[:, 3]


def _tc_proj(x, Wm, Av, b, apply_elu):
    d_in = x.shape[1]
    grid = NPAD // BN
    vec = jax.ShapeDtypeStruct((NPAD,), jnp.float32)
    return pl.pallas_call(
        functools.partial(_tc_proj_kernel, apply_elu=apply_elu),
        grid=(grid,),
        in_specs=[
            pl.BlockSpec((BN, d_in), lambda i: (i, 0)),
            pl.BlockSpec((d_in, F), lambda i: (0, 0)),
            pl.BlockSpec((F, 4), lambda i: (0, 0)),
            pl.BlockSpec((d_in,), lambda i: (0,)),
        ],
        out_specs=[
            pl.BlockSpec((BN, F), lambda i: (i, 0)),
            pl.BlockSpec((BN,), lambda i: (i,)),
            pl.BlockSpec((BN,), lambda i: (i,)),
            pl.BlockSpec((BN,), lambda i: (i,)),
            pl.BlockSpec((BN,), lambda i: (i,)),
        ],
        out_shape=[
            jax.ShapeDtypeStruct((NPAD, F), jnp.float32),
            vec, vec, vec, vec,
        ],
    )(x, Wm, Av, b)


def _pool_fc_kernel(h_ref, batch_ref, b2_ref, wfc_ref, bfc_ref, out_ref,
                    acc_ref, cnt_ref):
    i = pl.program_id(0)
    nb = pl.num_programs(0)

    @pl.when(i == 0)
    def _init():
        acc_ref[...] = jnp.zeros_like(acc_ref)
        cnt_ref[...] = jnp.zeros_like(cnt_ref)

    h = h_ref[...] + b2_ref[...][None, :]
    h = jnp.where(h > 0, h, jnp.exp(jnp.minimum(h, 0.0)) - 1.0)
    bt = batch_ref[...]
    onehot = (bt[:, None] == jax.lax.broadcasted_iota(jnp.int32, (BN, G), 1)
              ).astype(jnp.float32)
    acc_ref[...] += lax.dot_general(onehot, h, (((0,), (0,)), ((), ())),
                                    preferred_element_type=jnp.float32)
    cnt_ref[...] += jnp.sum(onehot, axis=0)

    @pl.when(i == nb - 1)
    def _fin():
        pooled = acc_ref[...] / jnp.maximum(cnt_ref[...], 1.0)[:, None]
        out_ref[...] = (
            lax.dot_general(pooled, wfc_ref[...], (((1,), (1,)), ((), ())),
                            preferred_element_type=jnp.float32)
            + bfc_ref[...][None, :]
        )


def _pool_fc(h, batch_p, b2, Wfc, bfc):
    grid = NPAD // BN
    return pl.pallas_call(
        _pool_fc_kernel,
        grid=(grid,),
        in_specs=[
            pl.BlockSpec((BN, F), lambda i: (i, 0)),
            pl.BlockSpec((BN,), lambda i: (i,)),
            pl.BlockSpec((F,), lambda i: (0,)),
            pl.BlockSpec((EMBED, F), lambda i: (0, 0)),
            pl.BlockSpec((EMBED,), lambda i: (0,)),
        ],
        out_specs=pl.BlockSpec((G, EMBED), lambda i: (0, 0)),
        out_shape=jax.ShapeDtypeStruct((G, EMBED), jnp.float32),
        scratch_shapes=[
            pltpu.VMEM((G, F), jnp.float32),
            pltpu.VMEM((G,), jnp.float32),
        ],
    )(h, batch_p, b2, Wfc, bfc)


def _att_mat(a_s, a_d):
    z = jnp.zeros((HID,), jnp.float32)
    c0 = jnp.concatenate([a_s[0], z])
    c1 = jnp.concatenate([z, a_s[1]])
    c2 = jnp.concatenate([a_d[0], z])
    c3 = jnp.concatenate([z, a_d[1]])
    return jnp.stack([c0, c1, c2, c3], axis=1)


def kernel(x, edge_index, batch, W1, a1_src, a1_dst, b1, W2, a2_src, a2_dst, b2,
           Wfc, bfc):
    ei = edge_index.astype(jnp.int32)
    loop = jnp.arange(N, dtype=jnp.int32)
    pad_e = E2 - (E + N)
    src = jnp.concatenate([ei[0], loop, jnp.zeros((pad_e,), jnp.int32)])
    dst = jnp.concatenate([ei[1], loop, jnp.full((pad_e,), NPAD, jnp.int32)])

    counts = _phase_hist(dst)
    cur, off, tot = _phase_prefix(counts)
    ebuf = _phase_scatter(src, dst, cur)

    xp = jnp.concatenate([x, jnp.zeros((NPAD - N, IN_DIM), jnp.float32)], axis=0)
    h1, s10, s11, d10, d11 = _tc_proj(xp, W1, _att_mat(a1_src, a1_dst),
                                      jnp.zeros((IN_DIM,), jnp.float32),
                                      apply_elu=False)
    o1 = _edge_pass(h1, s10, s11, d10, d11, ebuf, off, tot)
    h2, s20, s21, d20, d21 = _tc_proj(o1, W2, _att_mat(a2_src, a2_dst), b1,
                                      apply_elu=True)
    o2 = _edge_pass(h2, s20, s21, d20, d21, ebuf, off, tot)

    batch_p = jnp.concatenate([batch.astype(jnp.int32),
                               jnp.full((NPAD - N,), G, jnp.int32)])
    return _pool_fc(o2, batch_p, b2, Wfc, bfc)


# speedup vs baseline: 1.0043x; 1.0043x over previous
import functools

import jax
import jax.numpy as jnp
from jax import lax
from jax.experimental import pallas as pl
from jax.experimental.pallas import tpu as pltpu
from jax.experimental.pallas import tpu_sc as plsc

N = 100000
E = 1600000
IN_DIM = 4
HID = 64
HEADS = 2
EMBED = 64
G = 64
F = HEADS * HID

R = 512
NB = 196
NBP = 272
NPAD = NB * R
BN = 2048
E2 = 1703936
ECH = 1024
NCH = E2 // ECH
W = 32
CPT = NCH // W
K = 128
EPAD = E2 + NBP * K + ECH
BPT = 7
AW = 144

_mesh = plsc.VectorSubcoreMesh(core_axis_name="c", subcore_axis_name="s")


def _z16i():
    return jnp.zeros((16,), jnp.int32)


def _z16f():
    return jnp.zeros((16,), jnp.float32)


def _wid():
    return lax.axis_index("s") * 2 + lax.axis_index("c")


def _lanes():
    return lax.iota(jnp.int32, 16)


@functools.partial(
    pl.kernel, mesh=_mesh,
    out_type=jax.ShapeDtypeStruct((W * NBP,), jnp.int32),
    scratch_types=[pltpu.VMEM((ECH,), jnp.int32), pltpu.VMEM((NBP,), jnp.int32)],
)
def _phase_hist(dst_hbm, counts_hbm, dbuf, cntb):
    wid = _wid()
    lanes = _lanes()
    one0 = jnp.where(lanes == 0, 1, 0)
    for j in range(NBP // 16):
        cntb[pl.ds(j * 16, 16)] = _z16i()

    def chunk(c, _):
        pltpu.sync_copy(dst_hbm.at[pl.ds((wid * CPT + c) * ECH, ECH)], dbuf)

        def grp(g, _):
            bv = dbuf[pl.ds(g * 16, 16)] >> 9
            for l in range(16):
                plsc.addupdate(cntb.at[pl.ds(bv[l], 16)], one0)
            return 0
        return lax.fori_loop(0, ECH // 16, grp, 0)
    lax.fori_loop(0, CPT, chunk, 0)
    pltpu.sync_copy(cntb, counts_hbm.at[pl.ds(pl.multiple_of(wid * NBP, 16), NBP)])


@functools.partial(
    pl.kernel, mesh=_mesh,
    out_type=(
        jax.ShapeDtypeStruct((W * NBP,), jnp.int32),
        jax.ShapeDtypeStruct((NBP,), jnp.int32),
        jax.ShapeDtypeStruct((NBP,), jnp.int32),
    ),
    scratch_types=[
        pltpu.VMEM((W * NBP,), jnp.int32),
        pltpu.VMEM((W * NBP,), jnp.int32),
        pltpu.VMEM((NBP,), jnp.int32),
        pltpu.VMEM((NBP,), jnp.int32),
        pltpu.VMEM((NBP,), jnp.int32),
    ],
)
def _phase_prefix(counts_hbm, cur_hbm, off_hbm, tot_hbm, cnts, curb, offb, totb,
                  runb):
    @pl.when(_wid() == 0)
    def _():
        pltpu.sync_copy(counts_hbm, cnts)
        nj = NBP // 16
        for j in range(nj):
            runb[pl.ds(j * 16, 16)] = _z16i()

        def tile(t, _):
            tb = t * NBP
            for j in range(nj):
                r = runb[pl.ds(j * 16, 16)]
                curb[pl.ds(tb + j * 16, 16)] = r
                runb[pl.ds(j * 16, 16)] = r + cnts[pl.ds(tb + j * 16, 16)]
            return 0
        lax.fori_loop(0, W, tile, 0)

        lanes = _lanes()
        carry = jnp.int32(0)
        for j in range(nj):
            tot = runb[pl.ds(j * 16, 16)]
            padded = (tot + (K - 1)) & ~(K - 1)
            excl = _z16i()
            for l in range(16):
                excl = jnp.where(lanes == l, carry, excl)
                carry = carry + padded[l]
            offb[pl.ds(j * 16, 16)] = excl
            totb[pl.ds(j * 16, 16)] = tot

        def tile2(t, _):
            tb = t * NBP
            for j in range(nj):
                curb[pl.ds(tb + j * 16, 16)] = (
                    curb[pl.ds(tb + j * 16, 16)] + offb[pl.ds(j * 16, 16)])
            return 0
        lax.fori_loop(0, W, tile2, 0)
        pltpu.sync_copy(curb, cur_hbm)
        pltpu.sync_copy(offb, off_hbm)
        pltpu.sync_copy(totb, tot_hbm)


@functools.partial(
    pl.kernel, mesh=_mesh,
    out_type=jax.ShapeDtypeStruct((EPAD,), jnp.int32),
    scratch_types=[
        pltpu.VMEM((ECH,), jnp.int32),
        pltpu.VMEM((ECH,), jnp.int32),
        pltpu.VMEM((NBP,), jnp.int32),
        pltpu.VMEM((K,), jnp.int32),
        pltpu.VMEM((K,), jnp.int32),
        pltpu.VMEM((K,), jnp.int32),
        pltpu.VMEM((K,), jnp.int32),
        pltpu.SemaphoreType.DMA,
        pltpu.SemaphoreType.DMA,
    ],
)
def _phase_scatter(src_hbm, dst_hbm, cur_hbm, ebuf_hbm, sbuf, dbuf, curb,
                   idxw0, idxw1, valw0, valw1, semc0, semc1):
    idxwL = (idxw0, idxw1)
    valwL = (valw0, valw1)
    semcL = (semc0, semc1)
    wid = _wid()
    lanes = _lanes()
    one0 = jnp.where(lanes == 0, 1, 0)
    pltpu.sync_copy(cur_hbm.at[pl.ds(pl.multiple_of(wid * NBP, 16), NBP)], curb)

    def chunk(c, _):
        base = (wid * CPT + c) * ECH
        pltpu.sync_copy(src_hbm.at[pl.ds(base, ECH)], sbuf)
        pltpu.sync_copy(dst_hbm.at[pl.ds(base, ECH)], dbuf)

        def pairg(p, _):
            for par in range(2):
                g = p * 2 + par

                @pl.when((c > 0) | (p > 0))
                def _(par=par):
                    pltpu.make_async_copy(
                        valwL[par], ebuf_hbm.at[idxwL[par]], semcL[par]).wait()

                def sub(h, _, g=g, par=par):
                    ii = g * K + h * 16
                    dv = dbuf[pl.ds(ii, 16)]
                    sv = sbuf[pl.ds(ii, 16)]
                    bv = dv >> 9
                    vv = (sv << 9) | (dv & (R - 1))
                    db = _z16i()
                    for m in range(15):
                        db = db + jnp.where((bv == bv[m]) & (lanes > m), 1, 0)
                    cbase = _z16i()
                    for l in range(16):
                        cbase = jnp.where(lanes == l,
                                          curb[pl.ds(bv[l], 16)][0], cbase)
                    for l in range(16):
                        plsc.addupdate(curb.at[pl.ds(bv[l], 16)], one0)
                    idxwL[par][pl.ds(h * 16, 16)] = cbase + db
                    valwL[par][pl.ds(h * 16, 16)] = vv
                    return 0
                lax.fori_loop(0, K // 16, sub, 0)
                pltpu.async_copy(valwL[par], ebuf_hbm.at[idxwL[par]],
                                 semcL[par])
            return 0
        return lax.fori_loop(0, ECH // K // 2, pairg, 0)
    lax.fori_loop(0, CPT, chunk, 0)
    for par in range(2):
        pltpu.make_async_copy(
            valwL[par], ebuf_hbm.at[idxwL[par]], semcL[par]).wait()


@functools.partial(
    pl.kernel, mesh=_mesh,
    out_type=jax.ShapeDtypeStruct((NPAD, F), jnp.float32),
    scratch_types=[
        pltpu.VMEM((R * AW,), jnp.float32),
        pltpu.VMEM((K, F), jnp.float32),
        pltpu.VMEM((K, F), jnp.float32),
        pltpu.VMEM((ECH,), jnp.int32),
        pltpu.VMEM((K,), jnp.int32),
        pltpu.VMEM((K,), jnp.int32),
        pltpu.VMEM((K,), jnp.int32),
        pltpu.VMEM((K,), jnp.int32),
        pltpu.VMEM((K + 16,), jnp.int32),
        pltpu.VMEM((K + 16,), jnp.int32),
        pltpu.VMEM((K,), jnp.float32),
        pltpu.VMEM((K,), jnp.float32),
        pltpu.VMEM((K,), jnp.float32),
        pltpu.VMEM((K,), jnp.float32),
        pltpu.VMEM((K,), jnp.float32),
        pltpu.VMEM((K,), jnp.float32),
        pltpu.VMEM((K,), jnp.float32),
        pltpu.VMEM((K,), jnp.float32),
        pltpu.VMEM((K + 16,), jnp.float32),
        pltpu.VMEM((K + 16,), jnp.float32),
        pltpu.VMEM((NBP,), jnp.int32),
        pltpu.VMEM((NBP,), jnp.int32),
        pltpu.SemaphoreType.DMA,
        pltpu.SemaphoreType.DMA,
        pltpu.SemaphoreType.DMA,
        pltpu.SemaphoreType.DMA,
    ],
)
def _edge_pass(h_hbm, as0_hbm, as1_hbm, ad0_hbm, ad1_hbm, ebuf_hbm, off_hbm,
               tot_hbm, out_hbm,
               acc, rows0, rows1, pks, ix0, ix1, dg0, dg1, dl0, dl1,
               sa00, sa01, sa10, sa11, da00, da01, da10, da11,
               ee0b, ee1b, offv, totv, semr0, semr1, sems0, sems1):
    rowsL = (rows0, rows1)
    ixL = (ix0, ix1)
    dgL = (dg0, dg1)
    dlL = (dl0, dl1)
    as0L = (sa00, sa01)
    as1L = (sa10, sa11)
    ad0L = (da00, da01)
    ad1L = (da10, da11)
    semrL = (semr0, semr1)
    semsL = (sems0, sems1)
    wid = _wid()
    lanes = _lanes()
    pltpu.sync_copy(off_hbm, offv)
    pltpu.sync_copy(tot_hbm, totv)

    def bucket_iter(bi, _):
        b = wid + W * bi

        @pl.when(b < NB)
        def _():
            base = pl.multiple_of(b * R, R)

            def zero(k, _):
                acc[pl.ds(k * 16, 16)] = _z16f()
                return 0
            lax.fori_loop(0, R * AW // 16, zero, 0, unroll=8)

            off = offv[pl.ds(b, 16)][0]
            cnt = totv[pl.ds(b, 16)][0]
            nb_batches = (cnt + K - 1) >> 7

            def fetch(jb, par):
                @pl.when((jb & 7) == 0)
                def _():
                    pltpu.sync_copy(
                        ebuf_hbm.at[pl.ds(
                            pl.multiple_of(off + jb * K, ECH), ECH)], pks)
                sb = (jb & 7) * K
                for g in range(K // 16):
                    pkv = pks[pl.ds(sb + g * 16, 16)]
                    dl16 = pkv & (R - 1)
                    ixL[par][pl.ds(g * 16, 16)] = pkv >> 9
                    dlL[par][pl.ds(g * 16, 16)] = dl16
                    dgL[par][pl.ds(g * 16, 16)] = dl16 + base
                pltpu.async_copy(h_hbm.at[ixL[par]], rowsL[par], semrL[par])
                pltpu.async_copy(as0_hbm.at[ixL[par]], as0L[par], semsL[par])
                pltpu.async_copy(as1_hbm.at[ixL[par]], as1L[par], semsL[par])
                pltpu.async_copy(ad0_hbm.at[dgL[par]], ad0L[par], semsL[par])
                pltpu.async_copy(ad1_hbm.at[dgL[par]], ad1L[par], semsL[par])

            def waitb(par):
                pltpu.make_async_copy(
                    h_hbm.at[ixL[par]], rowsL[par], semrL[par]).wait()
                pltpu.make_async_copy(
                    as0_hbm.at[ixL[par]], as0L[par], semsL[par]).wait()
                pltpu.make_async_copy(
                    as1_hbm.at[ixL[par]], as1L[par], semsL[par]).wait()
                pltpu.make_async_copy(
                    ad0_hbm.at[dgL[par]], ad0L[par], semsL[par]).wait()
                pltpu.make_async_copy(
                    ad1_hbm.at[dgL[par]], ad1L[par], semsL[par]).wait()

            def compute(jb, par):
                rows = rowsL[par]
                dlv = dlL[par]
                for g in range(K // 16):
                    lv = lanes + g * 16
                    valid = (jb * K + lv) < cnt
                    e0 = (as0L[par][pl.ds(g * 16, 16)]
                          + ad0L[par][pl.ds(g * 16, 16)])
                    e1 = (as1L[par][pl.ds(g * 16, 16)]
                          + ad1L[par][pl.ds(g * 16, 16)])
                    e0 = jnp.where(e0 > 0, e0, 0.2 * e0)
                    e1 = jnp.where(e1 > 0, e1, 0.2 * e1)
                    ee0b[pl.ds(g * 16, 16)] = jnp.where(valid, jnp.exp(e0), 0.0)
                    ee1b[pl.ds(g * 16, 16)] = jnp.where(valid, jnp.exp(e1), 0.0)

                def edge(e, _):
                    dl = dlv[pl.ds(e, 16)][0]
                    s0 = ee0b[pl.ds(e, 16)][0]
                    s1 = ee1b[pl.ds(e, 16)][0]
                    ebase = dl * AW
                    vs = [rows[e, pl.ds(j * 16, 16)] for j in range(F // 16)]
                    ms = [vs[j] * (s0 if j < 4 else s1) for j in range(F // 16)]
                    for j in range(F // 16):
                        plsc.addupdate(acc.at[pl.ds(ebase + j * 16, 16)], ms[j])
                    wv = jnp.where(lanes == 0, s0,
                                   jnp.where(lanes == 1, s1, 0.0))
                    plsc.addupdate(acc.at[pl.ds(ebase + F, 16)], wv)
                    return 0
                lax.fori_loop(0, K, edge, 0, unroll=2)

            @pl.when(nb_batches > 0)
            def _prime():
                fetch(0, 0)

            def pair(jp, _):
                for par in range(2):
                    jb = jp * 2 + par

                    @pl.when(jb < nb_batches)
                    def _(jb=jb, par=par):
                        waitb(par)

                        @pl.when(jb + 1 < nb_batches)
                        def _():
                            fetch(jb + 1, 1 - par)
                        compute(jb, par)
                return 0
            lax.fori_loop(0, (nb_batches + 1) >> 1, pair, 0)

            def flush_chunk(cc, _):
                def node(rr, _):
                    nn = cc * K + rr
                    wv = acc[pl.ds(nn * AW + F, 16)]
                    inv = 1.0 / jnp.maximum(wv, 1e-30)
                    i0 = inv[0]
                    i1 = inv[1]
                    for j in range(F // 16):
                        s = i0 if j < 4 else i1
                        rows0[rr, pl.ds(j * 16, 16)] = (
                            acc[pl.ds(nn * AW + j * 16, 16)] * s)
                    return 0
                lax.fori_loop(0, K, node, 0, unroll=2)
                pltpu.sync_copy(
                    rows0,
                    out_hbm.at[pl.ds(pl.multiple_of(base + cc * K, K), K)])
                return 0
            lax.fori_loop(0, R // K, flush_chunk, 0)
        return 0
    lax.fori_loop(0, BPT, bucket_iter, 0)


def _tc_proj_kernel(x_ref, w_ref, av_ref, b_ref, h_ref, as0_ref, as1_ref,
                    ad0_ref, ad1_ref, *, apply_elu):
    x = x_ref[...]
    if apply_elu:
        x = x + b_ref[...][None, :]
        x = jnp.where(x > 0, x, jnp.exp(jnp.minimum(x, 0.0)) - 1.0)
    h = jnp.dot(x, w_ref[...], preferred_element_type=jnp.float32)
    h_ref[...] = h
    p = jnp.dot(h, av_ref[...], preferred_element_type=jnp.float32)
    as0_ref[...] = p[:, 0]
    as1_ref[...] = p[:, 1]
    ad0_ref[...] = p[:, 2]
    ad1_ref[...] = p[:, 3]


def _tc_proj(x, Wm, Av, b, apply_elu):
    d_in = x.shape[1]
    grid = NPAD // BN
    vec = jax.ShapeDtypeStruct((NPAD,), jnp.float32)
    return pl.pallas_call(
        functools.partial(_tc_proj_kernel, apply_elu=apply_elu),
        grid=(grid,),
        in_specs=[
            pl.BlockSpec((BN, d_in), lambda i: (i, 0)),
            pl.BlockSpec((d_in, F), lambda i: (0, 0)),
            pl.BlockSpec((F, 4), lambda i: (0, 0)),
            pl.BlockSpec((d_in,), lambda i: (0,)),
        ],
        out_specs=[
            pl.BlockSpec((BN, F), lambda i: (i, 0)),
            pl.BlockSpec((BN,), lambda i: (i,)),
            pl.BlockSpec((BN,), lambda i: (i,)),
            pl.BlockSpec((BN,), lambda i: (i,)),
            pl.BlockSpec((BN,), lambda i: (i,)),
        ],
        out_shape=[
            jax.ShapeDtypeStruct((NPAD, F), jnp.float32),
            vec, vec, vec, vec,
        ],
    )(x, Wm, Av, b)


def _pool_fc_kernel(h_ref, batch_ref, b2_ref, wfc_ref, bfc_ref, out_ref,
                    acc_ref, cnt_ref):
    i = pl.program_id(0)
    nb = pl.num_programs(0)

    @pl.when(i == 0)
    def _init():
        acc_ref[...] = jnp.zeros_like(acc_ref)
        cnt_ref[...] = jnp.zeros_like(cnt_ref)

    h = h_ref[...] + b2_ref[...][None, :]
    h = jnp.where(h > 0, h, jnp.exp(jnp.minimum(h, 0.0)) - 1.0)
    bt = batch_ref[...]
    onehot = (bt[:, None] == jax.lax.broadcasted_iota(jnp.int32, (BN, G), 1)
              ).astype(jnp.float32)
    acc_ref[...] += lax.dot_general(onehot, h, (((0,), (0,)), ((), ())),
                                    preferred_element_type=jnp.float32)
    cnt_ref[...] += jnp.sum(onehot, axis=0)

    @pl.when(i == nb - 1)
    def _fin():
        pooled = acc_ref[...] / jnp.maximum(cnt_ref[...], 1.0)[:, None]
        out_ref[...] = (
            lax.dot_general(pooled, wfc_ref[...], (((1,), (1,)), ((), ())),
                            preferred_element_type=jnp.float32)
            + bfc_ref[...][None, :]
        )


def _pool_fc(h, batch_p, b2, Wfc, bfc):
    grid = NPAD // BN
    return pl.pallas_call(
        _pool_fc_kernel,
        grid=(grid,),
        in_specs=[
            pl.BlockSpec((BN, F), lambda i: (i, 0)),
            pl.BlockSpec((BN,), lambda i: (i,)),
            pl.BlockSpec((F,), lambda i: (0,)),
            pl.BlockSpec((EMBED, F), lambda i: (0, 0)),
            pl.BlockSpec((EMBED,), lambda i: (0,)),
        ],
        out_specs=pl.BlockSpec((G, EMBED), lambda i: (0, 0)),
        out_shape=jax.ShapeDtypeStruct((G, EMBED), jnp.float32),
        scratch_shapes=[
            pltpu.VMEM((G, F), jnp.float32),
            pltpu.VMEM((G,), jnp.float32),
        ],
    )(h, batch_p, b2, Wfc, bfc)


def _att_mat(a_s, a_d):
    z = jnp.zeros((HID,), jnp.float32)
    c0 = jnp.concatenate([a_s[0], z])
    c1 = jnp.concatenate([z, a_s[1]])
    c2 = jnp.concatenate([a_d[0], z])
    c3 = jnp.concatenate([z, a_d[1]])
    return jnp.stack([c0, c1, c2, c3], axis=1)


def kernel(x, edge_index, batch, W1, a1_src, a1_dst, b1, W2, a2_src, a2_dst, b2,
           Wfc, bfc):
    ei = edge_index.astype(jnp.int32)
    loop = jnp.arange(N, dtype=jnp.int32)
    pad_e = E2 - (E + N)
    src = jnp.concatenate([ei[0], loop, jnp.zeros((pad_e,), jnp.int32)])
    dst = jnp.concatenate([ei[1], loop, jnp.full((pad_e,), NPAD, jnp.int32)])

    counts = _phase_hist(dst)
    cur, off, tot = _phase_prefix(counts)
    ebuf = _phase_scatter(src, dst, cur)

    xp = jnp.concatenate([x, jnp.zeros((NPAD - N, IN_DIM), jnp.float32)], axis=0)
    h1, s10, s11, d10, d11 = _tc_proj(xp, W1, _att_mat(a1_src, a1_dst),
                                      jnp.zeros((IN_DIM,), jnp.float32),
                                      apply_elu=False)
    o1 = _edge_pass(h1, s10, s11, d10, d11, ebuf, off, tot)
    h2, s20, s21, d20, d21 = _tc_proj(o1, W2, _att_mat(a2_src, a2_dst), b1,
                                      apply_elu=True)
    o2 = _edge_pass(h2, s20, s21, d20, d21, ebuf, off, tot)

    batch_p = jnp.concatenate([batch.astype(jnp.int32),
                               jnp.full((NPAD - N,), G, jnp.int32)])
    return _pool_fc(o2, batch_p, b2, Wfc, bfc)
